# baseline (speedup 1.0000x reference)
import functools

import jax
import jax.numpy as jnp
from jax import lax
from jax.experimental import pallas as pl
from jax.experimental.pallas import tpu as pltpu
from jax.experimental.pallas import tpu_sc as plsc

_ROWS = 16384
_DIM = 2048
_EXPERTS = 64
_TILE = 2048
_NW = 32
_TPW = _ROWS // _NW
_GROUPS = _TPW // 16


def _logits_kernel(x_ref, wt_ref, b_ref, out_ref):
    lg = lax.dot_general(wt_ref[...], x_ref[...],
                         (((1,), (1,)), ((), ())),
                         preferred_element_type=jnp.float32)
    out_ref[...] = lg + b_ref[...]


def _topk_sc_kernel(lg_hbm, out_hbm, lg_v, i1_v, i2_v):
    wid = lax.axis_index("s") * 2 + lax.axis_index("c")
    base = wid * _TPW
    pltpu.sync_copy(lg_hbm.at[:, pl.ds(base, _TPW)], lg_v)

    def body(g, carry):
        off = g * 16
        m1 = lg_v[0, pl.ds(off, 16)]
        i1 = jnp.zeros((16,), jnp.int32)
        m2 = jnp.full((16,), -jnp.inf, jnp.float32)
        i2 = jnp.zeros((16,), jnp.int32)
        for e in range(1, _EXPERTS):
            v = lg_v[e, pl.ds(off, 16)]
            ev = jnp.full((16,), e, jnp.int32)
            gt1 = v > m1
            gt2 = v > m2
            m2 = jnp.where(gt1, m1, jnp.where(gt2, v, m2))
            i2 = jnp.where(gt1, i1, jnp.where(gt2, ev, i2))
            m1 = jnp.where(gt1, v, m1)
            i1 = jnp.where(gt1, ev, i1)
        i1_v[pl.ds(off, 16)] = i1
        i2_v[pl.ds(off, 16)] = i2
        return carry

    lax.fori_loop(0, _GROUPS, body, 0)
    pltpu.sync_copy(i1_v, out_hbm.at[0, pl.ds(base, _TPW)])
    pltpu.sync_copy(i2_v, out_hbm.at[1, pl.ds(base, _TPW)])


@jax.jit
def kernel(x, W, b):
    wt = W.T
    b2 = b.reshape(_EXPERTS, 1)
    logits_t = pl.pallas_call(
        _logits_kernel,
        grid=(_ROWS // _TILE,),
        in_specs=[
            pl.BlockSpec((_TILE, _DIM), lambda i: (i, 0)),
            pl.BlockSpec((_EXPERTS, _DIM), lambda i: (0, 0)),
            pl.BlockSpec((_EXPERTS, 1), lambda i: (0, 0)),
        ],
        out_specs=pl.BlockSpec((_EXPERTS, _TILE), lambda i: (0, i)),
        out_shape=jax.ShapeDtypeStruct((_EXPERTS, _ROWS), jnp.float32),
        compiler_params=pltpu.CompilerParams(
            dimension_semantics=("parallel",),
        ),
    )(x, wt, b2)

    mesh = plsc.VectorSubcoreMesh(core_axis_name="c", subcore_axis_name="s")
    topk = pl.kernel(
        _topk_sc_kernel,
        mesh=mesh,
        out_type=jax.ShapeDtypeStruct((2, _ROWS), jnp.int32),
        scratch_types=[
            pltpu.VMEM((_EXPERTS, _TPW), jnp.float32),
            pltpu.VMEM((_TPW,), jnp.int32),
            pltpu.VMEM((_TPW,), jnp.int32),
        ],
    )(logits_t)
    return topk.T


# speedup vs baseline: 1.2892x; 1.2892x over previous
import functools

import jax
import jax.numpy as jnp
from jax import lax
from jax.experimental import pallas as pl
from jax.experimental.pallas import tpu as pltpu

_ROWS = 16384
_DIM = 2048
_EXPERTS = 64
_TILE = 2048


def _router_kernel(x_ref, wt_ref, b_ref, out_ref):
    logits = lax.dot_general(wt_ref[...], x_ref[...],
                             (((1,), (1,)), ((), ())),
                             preferred_element_type=jnp.float32)
    logits = logits + b_ref[...]
    t = logits.shape[1]
    iota = jax.lax.broadcasted_iota(
        jnp.int32, (_EXPERTS, t), 0).astype(jnp.float32)
    m1 = jnp.max(logits, axis=0, keepdims=True)
    i1 = jnp.min(jnp.where(logits == m1, iota, float(_EXPERTS)),
                 axis=0, keepdims=True)
    masked = jnp.where(iota == i1, -jnp.inf, logits)
    m2 = jnp.max(masked, axis=0, keepdims=True)
    i2 = jnp.min(jnp.where(masked == m2, iota, float(_EXPERTS)),
                 axis=0, keepdims=True)
    out_ref[...] = jnp.concatenate([i1, i2], axis=0).astype(jnp.int32).T


@jax.jit
def kernel(x, W, b):
    wt = W.T
    b2 = b.reshape(_EXPERTS, 1)
    grid = (_ROWS // _TILE,)
    out_t = pl.pallas_call(
        _router_kernel,
        grid=grid,
        in_specs=[
            pl.BlockSpec((_TILE, _DIM), lambda i: (i, 0)),
            pl.BlockSpec((_EXPERTS, _DIM), lambda i: (0, 0)),
            pl.BlockSpec((_EXPERTS, 1), lambda i: (0, 0)),
        ],
        out_specs=pl.BlockSpec((_TILE, 2), lambda i: (i, 0)),
        out_shape=jax.ShapeDtypeStruct((_ROWS, 2), jnp.int32),
        compiler_params=pltpu.CompilerParams(
            dimension_semantics=("parallel",),
        ),
    )(x, wt, b2)
    return out_t
